# baseline (speedup 1.0000x reference)
import functools

import jax
import jax.numpy as jnp
from jax import lax
from jax.experimental import pallas as pl
from jax.experimental.pallas import tpu as pltpu
from jax.experimental.pallas import tpu_sc as plsc

N = 16384
D = 32
K = 8192
BZ = 512
NB = N // BZ
_COMMIT = 0.25


def _round_f32_to_bf16(x):
    u = lax.bitcast_convert_type(x, jnp.int32)
    r = (u + jnp.int32(0x7FFF) + ((u >> 16) & jnp.int32(1))) & jnp.int32(-65536)
    return lax.bitcast_convert_type(r, jnp.float32)


def _tc_body(z2_ref, embt_ref, col_ref, idx_ref, loss_ref):
    z2 = z2_ref[...]
    embt = embt_ref[...]
    s2 = jnp.dot(z2, embt, preferred_element_type=jnp.float32)
    e_sq = jnp.sum(embt * embt, axis=0, keepdims=True)
    z_sq = 0.25 * jnp.sum(z2 * z2, axis=1, keepdims=True)
    dist = (z_sq + s2) + e_sq
    H = K // 2
    d1 = dist[:, :H]
    d2 = dist[:, H:]
    m1 = jnp.min(d1, axis=1, keepdims=True)
    m2 = jnp.min(d2, axis=1, keepdims=True)
    col = col_ref[...]
    fH = jnp.float32(H)
    i1 = jnp.min(jnp.where(d1 == m1, col, fH), axis=1)
    i2 = jnp.min(jnp.where(d2 == m2, col, fH), axis=1) + fH
    take2 = m2[:, 0] < _round_f32_to_bf16(m1)[:, 0]
    idx = jnp.where(take2, i2, i1)
    idx_ref[0, 0, :] = idx.astype(jnp.int32)
    mch = jnp.where(take2, m2[:, 0], m1[:, 0])
    loss_ref[0, 0, 0] = jnp.sum(mch)


def _tc_argmin(z_flat, emb_t):
    idx3, loss = pl.pallas_call(
        _tc_body,
        grid=(NB,),
        in_specs=[
            pl.BlockSpec((BZ, D), lambda i: (i, 0)),
            pl.BlockSpec((D, K), lambda i: (0, 0)),
            pl.BlockSpec((1, K // 2), lambda i: (0, 0)),
        ],
        out_specs=[
            pl.BlockSpec((1, 1, BZ), lambda i: (i, 0, 0)),
            pl.BlockSpec((1, 1, 1), lambda i: (i, 0, 0), memory_space=pltpu.SMEM),
        ],
        out_shape=[
            jax.ShapeDtypeStruct((NB, 1, BZ), jnp.int32),
            jax.ShapeDtypeStruct((NB, 1, 1), jnp.float32),
        ],
        compiler_params=pltpu.CompilerParams(
            dimension_semantics=("parallel",),
        ),
    )(z_flat, emb_t, jnp.arange(K // 2, dtype=jnp.float32).reshape(1, -1))
    return idx3.reshape(N), jnp.sum(loss) * (_COMMIT / (N * D))


_NC, _NS, _L = 2, 16, 16
_NW = _NC * _NS
_PER_W = N // _NW
_CH = 128
_NCH = _PER_W // _CH
_U_PER = N // _NS
_U_NCH = _U_PER // _CH
_SEG = K // _NS


def _sc_run(emb, indices):
    mesh = plsc.VectorSubcoreMesh(core_axis_name="c", subcore_axis_name="s")

    @functools.partial(
        pl.kernel,
        mesh=mesh,
        compiler_params=pltpu.CompilerParams(use_tc_tiling_on_sc=False),
        out_type=[
            jax.ShapeDtypeStruct((N, D), jnp.float32),
            jax.ShapeDtypeStruct((K,), jnp.float32),
        ],
        scratch_types=[
            pltpu.VMEM((_CH,), jnp.int32),
            pltpu.VMEM((_CH, D), jnp.float32),
            pltpu.VMEM((_CH,), jnp.float32),
            pltpu.VMEM((_SEG,), jnp.float32),
            pltpu.VMEM_SHARED((K,), jnp.float32),
            pltpu.SemaphoreType.DMA,
        ],
    )
    def sc_kernel(emb_hbm, idx_hbm, zq_hbm, usage_hbm,
                  idx_v, rows_v, ones_v, zeros_v, usage_sh, sem):
        c = lax.axis_index("c")
        s = lax.axis_index("s")
        wid = s * _NC + c

        @pl.when(c == 0)
        def _():
            def zbody(i, _):
                zeros_v[pl.ds(i * _L, _L)] = jnp.zeros((_L,), jnp.float32)
                return 0
            lax.fori_loop(0, _SEG // _L, zbody, 0)

            def obody(i, _):
                ones_v[pl.ds(i * _L, _L)] = jnp.ones((_L,), jnp.float32)
                return 0
            lax.fori_loop(0, _CH // _L, obody, 0)
            pltpu.sync_copy(zeros_v, usage_sh.at[pl.ds(s * _SEG, _SEG)])

        plsc.subcore_barrier()

        base = wid * _PER_W

        def gbody(j, _):
            off = base + j * _CH
            pltpu.sync_copy(idx_hbm.at[pl.ds(off, _CH)], idx_v)
            pltpu.async_copy(emb_hbm.at[idx_v], rows_v, sem).wait()
            pltpu.sync_copy(rows_v, zq_hbm.at[pl.ds(off, _CH)])
            return 0

        lax.fori_loop(0, _NCH, gbody, 0)

        @pl.when(c == 0)
        def _():
            ubase = s * _U_PER

            def ubody(j, _):
                pltpu.sync_copy(idx_hbm.at[pl.ds(ubase + j * _CH, _CH)], idx_v)
                pltpu.sync_copy(ones_v, usage_sh.at[idx_v], add=True)
                return 0

            lax.fori_loop(0, _U_NCH, ubody, 0)

        plsc.subcore_barrier()

        @pl.when(c == 0)
        def _():
            pltpu.sync_copy(usage_sh.at[pl.ds(s * _SEG, _SEG)],
                            usage_hbm.at[pl.ds(s * _SEG, _SEG)])

    return sc_kernel(emb, indices)


def kernel(z_flat, embeddings):
    emb_t = embeddings.T
    indices, loss = _tc_argmin(z_flat * -2.0, emb_t)
    z_q, usage = _sc_run(embeddings, indices)
    return (z_q, indices, loss, usage)


# speedup vs baseline: 1.0071x; 1.0071x over previous
import functools

import jax
import jax.numpy as jnp
from jax import lax
from jax.experimental import pallas as pl
from jax.experimental.pallas import tpu as pltpu
from jax.experimental.pallas import tpu_sc as plsc

N = 16384
D = 32
K = 8192
BZ = 512
NB = N // BZ
_COMMIT = 0.25


def _round_f32_to_bf16(x):
    u = lax.bitcast_convert_type(x, jnp.int32)
    r = (u + jnp.int32(0x7FFF) + ((u >> 16) & jnp.int32(1))) & jnp.int32(-65536)
    return lax.bitcast_convert_type(r, jnp.float32)


def _tc_body(z2_ref, embt_ref, idx_ref, loss_ref):
    z2 = z2_ref[...]
    embt = embt_ref[...]
    s2 = jnp.dot(z2, embt, preferred_element_type=jnp.float32)
    e_sq = jnp.sum(embt * embt, axis=0, keepdims=True)
    z_sq = 0.25 * jnp.sum(z2 * z2, axis=1, keepdims=True)
    dist = (z_sq + s2) + e_sq
    H = K // 2
    d1 = dist[:, :H]
    d2 = dist[:, H:]
    m1 = jnp.min(d1, axis=1, keepdims=True)
    m2 = jnp.min(d2, axis=1, keepdims=True)
    col = lax.broadcasted_iota(jnp.int32, (BZ, H), 1).astype(jnp.float32)
    fH = jnp.float32(H)
    i1 = jnp.min(jnp.where(d1 == m1, col, fH), axis=1)
    i2 = jnp.min(jnp.where(d2 == m2, col, fH), axis=1) + fH
    take2 = m2[:, 0] < _round_f32_to_bf16(m1)[:, 0]
    idx = jnp.where(take2, i2, i1)
    idx_ref[0, 0, :] = idx.astype(jnp.int32)
    mch = jnp.where(take2, m2[:, 0], m1[:, 0])
    loss_ref[0, 0, 0] = jnp.sum(mch)


def _tc_argmin(z_flat, emb_t):
    idx3, loss = pl.pallas_call(
        _tc_body,
        grid=(NB,),
        in_specs=[
            pl.BlockSpec((BZ, D), lambda i: (i, 0)),
            pl.BlockSpec((D, K), lambda i: (0, 0)),
        ],
        out_specs=[
            pl.BlockSpec((1, 1, BZ), lambda i: (i, 0, 0)),
            pl.BlockSpec((1, 1, 1), lambda i: (i, 0, 0), memory_space=pltpu.SMEM),
        ],
        out_shape=[
            jax.ShapeDtypeStruct((NB, 1, BZ), jnp.int32),
            jax.ShapeDtypeStruct((NB, 1, 1), jnp.float32),
        ],
        compiler_params=pltpu.CompilerParams(
            dimension_semantics=("parallel",),
        ),
    )(z_flat, emb_t)
    return idx3.reshape(N), jnp.sum(loss) * (_COMMIT / (N * D))


_NC, _NS, _L = 2, 16, 16
_NW = _NC * _NS
_PER_W = N // _NW
_CH = 128
_NCH = _PER_W // _CH
_U_PER = N // _NS
_U_NCH = _U_PER // _CH
_SEG = K // _NS


def _sc_run(emb, indices):
    mesh = plsc.VectorSubcoreMesh(core_axis_name="c", subcore_axis_name="s")

    @functools.partial(
        pl.kernel,
        mesh=mesh,
        compiler_params=pltpu.CompilerParams(use_tc_tiling_on_sc=False),
        out_type=[
            jax.ShapeDtypeStruct((N, D), jnp.float32),
            jax.ShapeDtypeStruct((K,), jnp.float32),
        ],
        scratch_types=[
            pltpu.VMEM((_CH,), jnp.int32),
            pltpu.VMEM((_CH, D), jnp.float32),
            pltpu.VMEM((_CH,), jnp.float32),
            pltpu.VMEM((_SEG,), jnp.float32),
            pltpu.VMEM_SHARED((K,), jnp.float32),
            pltpu.SemaphoreType.DMA,
        ],
    )
    def sc_kernel(emb_hbm, idx_hbm, zq_hbm, usage_hbm,
                  idx_v, rows_v, ones_v, zeros_v, usage_sh, sem):
        c = lax.axis_index("c")
        s = lax.axis_index("s")
        wid = s * _NC + c

        @pl.when(c == 0)
        def _():
            def zbody(i, _):
                zeros_v[pl.ds(i * _L, _L)] = jnp.zeros((_L,), jnp.float32)
                return 0
            lax.fori_loop(0, _SEG // _L, zbody, 0)

            def obody(i, _):
                ones_v[pl.ds(i * _L, _L)] = jnp.ones((_L,), jnp.float32)
                return 0
            lax.fori_loop(0, _CH // _L, obody, 0)
            pltpu.sync_copy(zeros_v, usage_sh.at[pl.ds(s * _SEG, _SEG)])

        plsc.subcore_barrier()

        base = wid * _PER_W

        def gbody(j, _):
            off = base + j * _CH
            pltpu.sync_copy(idx_hbm.at[pl.ds(off, _CH)], idx_v)
            pltpu.async_copy(emb_hbm.at[idx_v], rows_v, sem).wait()
            pltpu.sync_copy(rows_v, zq_hbm.at[pl.ds(off, _CH)])
            return 0

        lax.fori_loop(0, _NCH, gbody, 0)

        @pl.when(c == 0)
        def _():
            ubase = s * _U_PER

            def ubody(j, _):
                pltpu.sync_copy(idx_hbm.at[pl.ds(ubase + j * _CH, _CH)], idx_v)
                pltpu.sync_copy(ones_v, usage_sh.at[idx_v], add=True)
                return 0

            lax.fori_loop(0, _U_NCH, ubody, 0)

        plsc.subcore_barrier()

        @pl.when(c == 0)
        def _():
            pltpu.sync_copy(usage_sh.at[pl.ds(s * _SEG, _SEG)],
                            usage_hbm.at[pl.ds(s * _SEG, _SEG)])

    return sc_kernel(emb, indices)


def kernel(z_flat, embeddings):
    emb_t = embeddings.T
    indices, loss = _tc_argmin(z_flat * -2.0, emb_t)
    z_q, usage = _sc_run(embeddings, indices)
    return (z_q, indices, loss, usage)
